# baseline (speedup 1.0000x reference)
import functools

import jax
import jax.numpy as jnp
from jax import lax
from jax.experimental import pallas as pl
from jax.experimental.pallas import tpu as pltpu
from jax.experimental.pallas import tpu_sc as plsc

B = 4096
L = 200
H = 64
LH = L // 2
EPS = 1e-5
NC = 2
NS = 16
NW = NC * NS
ROWS_PER_W = B // NW
NG = (L + 15) // 16
LP = NG * 16
NBUF = 4


def _rsqrt(x):
    i = plsc.bitcast(x, jnp.int32)
    i = 0x5F3759DF - (i >> 1)
    y = plsc.bitcast(i, jnp.float32)
    for _ in range(3):
        y = y * (1.5 - 0.5 * x * y * y)
    return y


def kernel(input_ids, word_emb, pos_emb, gamma, beta):
    ids2 = input_ids.reshape(2 * B, LH).astype(jnp.int32)
    mesh = plsc.VectorSubcoreMesh(core_axis_name="c", subcore_axis_name="s")

    @functools.partial(
        pl.kernel,
        out_type=jax.ShapeDtypeStruct((B, L, H), jnp.float32),
        mesh=mesh,
        compiler_params=pltpu.CompilerParams(
            needs_layout_passes=False, use_tc_tiling_on_sc=False),
        scratch_types=[
            pltpu.VMEM((2 * ROWS_PER_W, LH), jnp.int32),
            pltpu.VMEM((NBUF, L, H), jnp.float32),
            pltpu.VMEM((H, LP), jnp.float32),
            pltpu.VMEM((H,), jnp.float32),
            pltpu.VMEM((H,), jnp.float32),
            pltpu.VMEM((H, 16), jnp.float32),
            pltpu.VMEM((H, 16), jnp.float32),
            pltpu.VMEM((H, 16), jnp.float32),
            pltpu.VMEM((2, L, H), jnp.float32),
            pltpu.SemaphoreType.DMA,
            pltpu.SemaphoreType.DMA,
            pltpu.SemaphoreType.DMA,
            pltpu.SemaphoreType.DMA,
            pltpu.SemaphoreType.DMA,
            pltpu.SemaphoreType.DMA,
        ],
    )
    def run(ids_hbm, wemb_hbm, pemb_hbm, gamma_hbm, beta_hbm, out_hbm,
            idx_v, rows_v, pos_t, g_v, b_v, x_t, g_rot, b_rot, out_v,
            gsem0, gsem1, gsem2, gsem3, osem0, osem1):
        gsems = [gsem0, gsem1, gsem2, gsem3]
        osems = [osem0, osem1]
        wid = lax.axis_index("s") * NC + lax.axis_index("c")
        row0 = wid * ROWS_PER_W
        pltpu.sync_copy(ids_hbm.at[pl.ds(2 * row0, 2 * ROWS_PER_W)], idx_v)
        pltpu.sync_copy(gamma_hbm, g_v)
        pltpu.sync_copy(beta_hbm, b_v)
        iota = lax.iota(jnp.int32, 16)

        def ecol(h):
            return (iota + h) & (H - 1)

        def build_rot(h, carry):
            g_rot[h, pl.ds(0, 16)] = plsc.load_gather(g_v, [ecol(h)])
            b_rot[h, pl.ds(0, 16)] = plsc.load_gather(b_v, [ecol(h)])
            return carry

        lax.fori_loop(0, H, build_rot, 0)

        pltpu.sync_copy(pemb_hbm.at[pl.ds(0, L)], rows_v.at[0])

        def transpose_pos(g, carry):
            tok = jnp.minimum(g * 16 + iota, L - 1)

            def tp_h(h, c):
                pos_t[h, pl.ds(g * 16, 16)] = plsc.load_gather(
                    rows_v.at[0], [tok, ecol(h)])
                return c

            lax.fori_loop(0, H, tp_h, 0)
            return carry

        lax.fori_loop(0, NG, transpose_pos, 0)

        def issue_gather(r, buf):
            pltpu.async_copy(
                wemb_hbm.at[idx_v.at[2 * r]],
                rows_v.at[buf, pl.ds(0, LH)], gsems[buf])
            pltpu.async_copy(
                wemb_hbm.at[idx_v.at[2 * r + 1]],
                rows_v.at[buf, pl.ds(LH, LH)], gsems[buf])

        def wait_gather(r, buf):
            for j in range(2):
                pltpu.make_async_copy(
                    wemb_hbm.at[idx_v.at[2 * r + j]],
                    rows_v.at[buf, pl.ds(j * LH, LH)], gsems[buf]).wait()

        def compute_row(r, buf, obuf):
            def grp_body(g, carry):
                tok = jnp.minimum(g * 16 + iota, L - 1)
                zero = jnp.zeros((16,), jnp.float32)

                @plsc.parallel_loop(0, H, step=2, unroll=4,
                                    carry=(zero, zero, zero, zero))
                def pass1(h, c):
                    s0, q0, s1, q1 = c
                    w0 = plsc.load_gather(rows_v.at[buf], [tok, ecol(h)])
                    x0 = w0 + pos_t[h, pl.ds(g * 16, 16)]
                    x_t[h, pl.ds(0, 16)] = x0
                    w1 = plsc.load_gather(rows_v.at[buf], [tok, ecol(h + 1)])
                    x1 = w1 + pos_t[h + 1, pl.ds(g * 16, 16)]
                    x_t[h + 1, pl.ds(0, 16)] = x1
                    return (s0 + x0, q0 + x0 * x0, s1 + x1, q1 + x1 * x1)

                s0, q0, s1, q1 = pass1
                mean = (s0 + s1) * (1.0 / H)
                var = (q0 + q1) * (1.0 / H) - mean * mean
                inv = _rsqrt(var + EPS)

                @plsc.parallel_loop(0, H, step=2, unroll=8)
                def pass2(h):
                    for j in range(2):
                        x = x_t[h + j, pl.ds(0, 16)]
                        y = (x - mean) * inv
                        y = (y * g_rot[h + j, pl.ds(0, 16)]
                             + b_rot[h + j, pl.ds(0, 16)])
                        plsc.store_scatter(
                            out_v.at[obuf], [tok, ecol(h + j)], y)
                return carry

            lax.fori_loop(0, NG, grp_body, 0)

        def wait_out(r, obuf):
            pltpu.make_async_copy(
                out_v.at[obuf], out_hbm.at[row0 + r], osems[obuf]).wait()

        def issue_out(r, obuf):
            pltpu.async_copy(out_v.at[obuf], out_hbm.at[row0 + r], osems[obuf])

        for b in range(NBUF - 1):
            issue_gather(b, b)

        def quad_body(i, carry):
            for b in range(NBUF):
                r = NBUF * i + b
                nb = (b + NBUF - 1) % NBUF

                @pl.when(r + NBUF - 1 < ROWS_PER_W)
                def _():
                    issue_gather(r + NBUF - 1, nb)
                wait_gather(r, b)
                ob = b % 2

                @pl.when(r > 1)
                def _():
                    wait_out(r - 2, ob)
                compute_row(r, b, ob)
                issue_out(r, ob)
            return carry

        lax.fori_loop(0, ROWS_PER_W // NBUF, quad_body, 0)
        wait_out(ROWS_PER_W - 2, 0)
        wait_out(ROWS_PER_W - 1, 1)

    return run(ids2, word_emb, pos_emb, gamma, beta)


# speedup vs baseline: 1.0181x; 1.0181x over previous
import functools

import jax
import jax.numpy as jnp
from jax import lax
from jax.experimental import pallas as pl
from jax.experimental.pallas import tpu as pltpu
from jax.experimental.pallas import tpu_sc as plsc

B = 4096
L = 200
H = 64
LH = L // 2
EPS = 1e-5
NC = 2
NS = 16
NW = NC * NS
ROWS_PER_W = B // NW
NG = (L + 15) // 16
LP = NG * 16
NBUF = 4


def _rsqrt(x):
    i = plsc.bitcast(x, jnp.int32)
    i = 0x5F3759DF - (i >> 1)
    y = plsc.bitcast(i, jnp.float32)
    for _ in range(3):
        y = y * (1.5 - 0.5 * x * y * y)
    return y


def kernel(input_ids, word_emb, pos_emb, gamma, beta):
    ids2 = input_ids.reshape(2 * B, LH).astype(jnp.int32)
    mesh = plsc.VectorSubcoreMesh(core_axis_name="c", subcore_axis_name="s")

    @functools.partial(
        pl.kernel,
        out_type=jax.ShapeDtypeStruct((B, L, H), jnp.float32),
        mesh=mesh,
        compiler_params=pltpu.CompilerParams(
            needs_layout_passes=False, use_tc_tiling_on_sc=False),
        scratch_types=[
            pltpu.VMEM((2 * ROWS_PER_W, LH), jnp.int32),
            pltpu.VMEM((NBUF, L, H), jnp.float32),
            pltpu.VMEM((H, LP), jnp.float32),
            pltpu.VMEM((H,), jnp.float32),
            pltpu.VMEM((H,), jnp.float32),
            pltpu.VMEM((H, 16), jnp.float32),
            pltpu.VMEM((H, 16), jnp.float32),
            pltpu.VMEM((H, 16), jnp.float32),
            pltpu.VMEM((2, L, H), jnp.float32),
            pltpu.SemaphoreType.DMA,
            pltpu.SemaphoreType.DMA,
            pltpu.SemaphoreType.DMA,
            pltpu.SemaphoreType.DMA,
            pltpu.SemaphoreType.DMA,
            pltpu.SemaphoreType.DMA,
        ],
    )
    def run(ids_hbm, wemb_hbm, pemb_hbm, gamma_hbm, beta_hbm, out_hbm,
            idx_v, rows_v, pos_t, g_v, b_v, x_t, g_rot, b_rot, out_v,
            gsem0, gsem1, gsem2, gsem3, osem0, osem1):
        gsems = [gsem0, gsem1, gsem2, gsem3]
        osems = [osem0, osem1]
        wid = lax.axis_index("s") * NC + lax.axis_index("c")
        row0 = wid * ROWS_PER_W
        pltpu.sync_copy(ids_hbm.at[pl.ds(2 * row0, 2 * ROWS_PER_W)], idx_v)
        pltpu.sync_copy(gamma_hbm, g_v)
        pltpu.sync_copy(beta_hbm, b_v)
        iota = lax.iota(jnp.int32, 16)

        def ecol(h):
            return (iota + h) & (H - 1)

        def build_rot(h, carry):
            g_rot[h, pl.ds(0, 16)] = plsc.load_gather(g_v, [ecol(h)])
            b_rot[h, pl.ds(0, 16)] = plsc.load_gather(b_v, [ecol(h)])
            return carry

        lax.fori_loop(0, H, build_rot, 0)

        pltpu.sync_copy(pemb_hbm.at[pl.ds(0, L)], rows_v.at[0])

        def transpose_pos(g, carry):
            tok = jnp.minimum(g * 16 + iota, L - 1)

            def tp_h(h, c):
                pos_t[h, pl.ds(g * 16, 16)] = plsc.load_gather(
                    rows_v.at[0], [tok, ecol(h)])
                return c

            lax.fori_loop(0, H, tp_h, 0)
            return carry

        lax.fori_loop(0, NG, transpose_pos, 0)

        def issue_gather(r, buf):
            pltpu.async_copy(
                wemb_hbm.at[idx_v.at[2 * r]],
                rows_v.at[buf, pl.ds(0, LH)], gsems[buf])
            pltpu.async_copy(
                wemb_hbm.at[idx_v.at[2 * r + 1]],
                rows_v.at[buf, pl.ds(LH, LH)], gsems[buf])

        def wait_gather(r, buf):
            for j in range(2):
                pltpu.make_async_copy(
                    wemb_hbm.at[idx_v.at[2 * r + j]],
                    rows_v.at[buf, pl.ds(j * LH, LH)], gsems[buf]).wait()

        def compute_row(r, buf, obuf):
            def grp_body(g, carry):
                tok = jnp.minimum(g * 16 + iota, L - 1)
                zero = jnp.zeros((16,), jnp.float32)

                @plsc.parallel_loop(0, H, unroll=8,
                                    carry=(zero,) * 8)
                def pass1(h, c):
                    s0, q0, s1, q1, s2, q2, s3, q3 = c
                    w = plsc.load_gather(rows_v.at[buf], [tok, ecol(h)])
                    x = w + pos_t[h, pl.ds(g * 16, 16)]
                    x_t[h, pl.ds(0, 16)] = x
                    return (s1, q1, s2, q2, s3, q3, s0 + x, q0 + x * x)

                s0, q0, s1, q1, s2, q2, s3, q3 = pass1
                mean = ((s0 + s1) + (s2 + s3)) * (1.0 / H)
                var = (((q0 + q1) + (q2 + q3)) * (1.0 / H)
                       - mean * mean)
                inv = _rsqrt(var + EPS)

                @plsc.parallel_loop(0, H, unroll=16)
                def pass2(h):
                    x = x_t[h, pl.ds(0, 16)]
                    y = (x - mean) * inv
                    y = y * g_rot[h, pl.ds(0, 16)] + b_rot[h, pl.ds(0, 16)]
                    plsc.store_scatter(out_v.at[obuf], [tok, ecol(h)], y)
                return carry

            lax.fori_loop(0, NG, grp_body, 0)

        def wait_out(r, obuf):
            pltpu.make_async_copy(
                out_v.at[obuf], out_hbm.at[row0 + r], osems[obuf]).wait()

        def issue_out(r, obuf):
            pltpu.async_copy(out_v.at[obuf], out_hbm.at[row0 + r], osems[obuf])

        for b in range(NBUF - 1):
            issue_gather(b, b)

        def quad_body(i, carry):
            for b in range(NBUF):
                r = NBUF * i + b
                nb = (b + NBUF - 1) % NBUF

                @pl.when(r + NBUF - 1 < ROWS_PER_W)
                def _():
                    issue_gather(r + NBUF - 1, nb)
                wait_gather(r, b)
                ob = b % 2

                @pl.when(r > 1)
                def _():
                    wait_out(r - 2, ob)
                compute_row(r, b, ob)
                issue_out(r, ob)
            return carry

        lax.fori_loop(0, ROWS_PER_W // NBUF, quad_body, 0)
        wait_out(ROWS_PER_W - 2, 0)
        wait_out(ROWS_PER_W - 1, 1)

    return run(ids2, word_emb, pos_emb, gamma, beta)


# speedup vs baseline: 1.0320x; 1.0137x over previous
import functools

import jax
import jax.numpy as jnp
from jax import lax
from jax.experimental import pallas as pl
from jax.experimental.pallas import tpu as pltpu
from jax.experimental.pallas import tpu_sc as plsc

B = 4096
L = 200
H = 64
LH = L // 2
EPS = 1e-5
NC = 2
NS = 16
NW = NC * NS
ROWS_PER_W = B // NW
NG = (L + 15) // 16
LP = NG * 16
NBUF = 4


def _rsqrt(x):
    i = plsc.bitcast(x, jnp.int32)
    i = 0x5F3759DF - (i >> 1)
    y = plsc.bitcast(i, jnp.float32)
    for _ in range(3):
        y = y * (1.5 - 0.5 * x * y * y)
    return y


def kernel(input_ids, word_emb, pos_emb, gamma, beta):
    ids2 = input_ids.reshape(2 * B, LH).astype(jnp.int32)
    mesh = plsc.VectorSubcoreMesh(core_axis_name="c", subcore_axis_name="s")

    @functools.partial(
        pl.kernel,
        out_type=jax.ShapeDtypeStruct((B, L, H), jnp.float32),
        mesh=mesh,
        compiler_params=pltpu.CompilerParams(
            needs_layout_passes=False, use_tc_tiling_on_sc=False),
        scratch_types=[
            pltpu.VMEM((2 * ROWS_PER_W, LH), jnp.int32),
            pltpu.VMEM((NBUF, L, H), jnp.float32),
            pltpu.VMEM((H, LP), jnp.float32),
            pltpu.VMEM((H,), jnp.float32),
            pltpu.VMEM((H,), jnp.float32),
            pltpu.VMEM((H, H), jnp.float32),
            pltpu.VMEM((H, 16), jnp.float32),
            pltpu.VMEM((H, 16), jnp.float32),
            pltpu.VMEM((2, L, H), jnp.float32),
            pltpu.SemaphoreType.DMA,
            pltpu.SemaphoreType.DMA,
            pltpu.SemaphoreType.DMA,
            pltpu.SemaphoreType.DMA,
            pltpu.SemaphoreType.DMA,
            pltpu.SemaphoreType.DMA,
        ],
    )
    def run(ids_hbm, wemb_hbm, pemb_hbm, gamma_hbm, beta_hbm, out_hbm,
            idx_v, rows_v, pos_t, g_v, b_v, x_t, g_rot, b_rot, out_v,
            gsem0, gsem1, gsem2, gsem3, osem0, osem1):
        gsems = [gsem0, gsem1, gsem2, gsem3]
        osems = [osem0, osem1]
        wid = lax.axis_index("s") * NC + lax.axis_index("c")
        row0 = wid * ROWS_PER_W
        pltpu.sync_copy(ids_hbm.at[pl.ds(2 * row0, 2 * ROWS_PER_W)], idx_v)
        pltpu.sync_copy(gamma_hbm, g_v)
        pltpu.sync_copy(beta_hbm, b_v)
        iota = lax.iota(jnp.int32, 16)

        def ecol(h):
            return (iota + h) & (H - 1)

        def build_rot(h, carry):
            g_rot[h, pl.ds(0, 16)] = plsc.load_gather(g_v, [ecol(h)])
            b_rot[h, pl.ds(0, 16)] = plsc.load_gather(b_v, [ecol(h)])
            return carry

        lax.fori_loop(0, H, build_rot, 0)

        pltpu.sync_copy(pemb_hbm.at[pl.ds(0, L)], rows_v.at[0])

        def transpose_pos(g, carry):
            tok = jnp.minimum(g * 16 + iota, L - 1)

            def tp_h(h, c):
                pos_t[h, pl.ds(g * 16, 16)] = plsc.load_gather(
                    rows_v.at[0], [tok, ecol(h)])
                return c

            lax.fori_loop(0, H, tp_h, 0)
            return carry

        lax.fori_loop(0, NG, transpose_pos, 0)

        def issue_gather(r, buf):
            pltpu.async_copy(
                wemb_hbm.at[idx_v.at[2 * r]],
                rows_v.at[buf, pl.ds(0, LH)], gsems[buf])
            pltpu.async_copy(
                wemb_hbm.at[idx_v.at[2 * r + 1]],
                rows_v.at[buf, pl.ds(LH, LH)], gsems[buf])

        def wait_gather(r, buf):
            for j in range(2):
                pltpu.make_async_copy(
                    wemb_hbm.at[idx_v.at[2 * r + j]],
                    rows_v.at[buf, pl.ds(j * LH, LH)], gsems[buf]).wait()

        def compute_row(r, buf, obuf):
            zero = jnp.zeros((16,), jnp.float32)

            def do_batch(g0, ng):
                toks = [jnp.minimum((g0 + k) * 16 + iota, L - 1)
                        for k in range(ng)]

                @plsc.parallel_loop(0, H, unroll=4, carry=(zero,) * (2 * ng))
                def pass1(h, c):
                    out = []
                    for k in range(ng):
                        w = plsc.load_gather(
                            rows_v.at[buf], [toks[k], ecol(h)])
                        x = w + pos_t[h, pl.ds((g0 + k) * 16, 16)]
                        x_t[h, pl.ds(16 * k, 16)] = x
                        out += [c[2 * k] + x, c[2 * k + 1] + x * x]
                    return tuple(out)

                acc = pass1
                means, invs = [], []
                for k in range(ng):
                    mean = acc[2 * k] * (1.0 / H)
                    var = acc[2 * k + 1] * (1.0 / H) - mean * mean
                    means.append(mean)
                    invs.append(_rsqrt(var + EPS))

                @plsc.parallel_loop(0, H, unroll=4)
                def pass2(h):
                    gr = g_rot[h, pl.ds(0, 16)]
                    br = b_rot[h, pl.ds(0, 16)]
                    for k in range(ng):
                        x = x_t[h, pl.ds(16 * k, 16)]
                        y = (x - means[k]) * invs[k]
                        y = y * gr + br
                        plsc.store_scatter(
                            out_v.at[obuf], [toks[k], ecol(h)], y)

            for g0 in range(0, NG - 1, 4):
                do_batch(g0, 4)
            do_batch(NG - 1, 1)

        def wait_out(r, obuf):
            pltpu.make_async_copy(
                out_v.at[obuf], out_hbm.at[row0 + r], osems[obuf]).wait()

        def issue_out(r, obuf):
            pltpu.async_copy(out_v.at[obuf], out_hbm.at[row0 + r], osems[obuf])

        for b in range(NBUF - 1):
            issue_gather(b, b)

        def quad_body(i, carry):
            for b in range(NBUF):
                r = NBUF * i + b
                nb = (b + NBUF - 1) % NBUF

                @pl.when(r + NBUF - 1 < ROWS_PER_W)
                def _():
                    issue_gather(r + NBUF - 1, nb)
                wait_gather(r, b)
                ob = b % 2

                @pl.when(r > 1)
                def _():
                    wait_out(r - 2, ob)
                compute_row(r, b, ob)
                issue_out(r, ob)
            return carry

        lax.fori_loop(0, ROWS_PER_W // NBUF, quad_body, 0)
        wait_out(ROWS_PER_W - 2, 0)
        wait_out(ROWS_PER_W - 1, 1)

    return run(ids2, word_emb, pos_emb, gamma, beta)
